# Initial kernel scaffold; baseline (speedup 1.0000x reference)
import jax
import jax.numpy as jnp
from jax.experimental import pallas as pl


def kernel(x):
    raise NotImplementedError("write your pallas kernel here")


# speedup vs baseline: 173.1047x; 173.1047x over previous
import functools

import jax
import jax.numpy as jnp
from jax.experimental import pallas as pl
from jax.experimental.pallas import tpu as pltpu


def _topk_mask_body(x_ref, o_ref, *, block_rows, seq_len, k, prefix):
    i = pl.program_id(0)
    a = jnp.abs(x_ref[...])
    ai = jax.lax.bitcast_convert_type(a, jnp.int32)
    lo = jnp.zeros((block_rows, 1), jnp.int32)
    for b in range(30, -1, -1):
        cand = lo | (1 << b)
        cnt = jnp.sum((ai >= cand).astype(jnp.int32), axis=1, keepdims=True)
        lo = jnp.where(cnt >= k, cand, lo)
    mask = (ai >= lo).astype(jnp.float32)
    rows = i * block_rows + jax.lax.broadcasted_iota(jnp.int32, (block_rows, 1), 0)
    is_prefix = (rows % seq_len) < prefix
    o_ref[...] = jnp.where(is_prefix, 1.0, mask)


def _topk_mask(x2d, seq_len, k, prefix, block_rows=256, interpret=False):
    rows, d = x2d.shape
    grid = (rows // block_rows,)
    return pl.pallas_call(
        functools.partial(
            _topk_mask_body,
            block_rows=block_rows,
            seq_len=seq_len,
            k=k,
            prefix=prefix,
        ),
        grid=grid,
        in_specs=[pl.BlockSpec((block_rows, d), lambda i: (i, 0))],
        out_specs=pl.BlockSpec((block_rows, d), lambda i: (i, 0)),
        out_shape=jax.ShapeDtypeStruct((rows, d), jnp.float32),
        interpret=interpret,
    )(x2d)


def kernel(x):
    b, s, d = x.shape
    half_seq = int(0.99 * s)
    prefix = s - half_seq
    k = int(d * 0.5)
    block_rows = 256 if (b * s) % 256 == 0 else 8
    out = _topk_mask(x.reshape(b * s, d), s, k, prefix, block_rows=block_rows)
    return out.reshape(b, s, d)
